# baseline (speedup 1.0000x reference)
import functools

import jax
import jax.numpy as jnp
from jax import lax
from jax.experimental import pallas as pl
from jax.experimental.pallas import tpu as pltpu
from jax.experimental.pallas import tpu_sc as plsc

N = 10000
K = 16
D = 128
ND = 3
L = 4
H = 64

NPAD = 10240
NE = NPAD * K

BN = 256
GRID = NPAD // BN

NW = 32
ROWS_W = NE // NW


def _sc_gather(table, idx, wrow, chunk):
    nch = ROWS_W // chunk
    mesh = plsc.VectorSubcoreMesh(core_axis_name="c", subcore_axis_name="s")

    @functools.partial(
        pl.kernel,
        mesh=mesh,
        out_type=jax.ShapeDtypeStruct((NE, wrow), jnp.float32),
        scratch_types=[
            pltpu.VMEM((chunk,), jnp.int32),
            pltpu.VMEM((chunk, wrow), jnp.float32),
            pltpu.SemaphoreType.DMA,
        ],
    )
    def gather_k(table_hbm, idx_hbm, out_hbm, idx_v, rows_v, sem):
        wid = lax.axis_index("s") * 2 + lax.axis_index("c")
        base = wid * ROWS_W
        for c in range(nch):
            off = base + c * chunk
            pltpu.sync_copy(idx_hbm.at[pl.ds(off, chunk)], idx_v)
            pltpu.async_copy(table_hbm.at[idx_v], rows_v, sem).wait()
            pltpu.sync_copy(rows_v, out_hbm.at[pl.ds(off, chunk)])

    return gather_k(table, idx)


def _pre_body(inp_ref, im_ref, igd_ref, ogd_ref, lw_ref, lb_ref,
              x0_ref, ing_ref, outg_ref):
    x = jnp.dot(inp_ref[...], lw_ref[...], preferred_element_type=jnp.float32)
    x0_ref[...] = jax.nn.gelu(x + lb_ref[...])
    im = im_ref[...]
    ing_ref[...] = im + igd_ref[...]
    outg_ref[...] = im + ogd_ref[...]


def _compact_body(pw_ref, pj_ref):
    for k in range(K):
        pj_ref[:, k * 8:(k + 1) * 8] = pw_ref[:, k * D:k * D + 8]


def _layer_body(og_ref, pj_ref, xj_ref, x_ref, k1a_ref, k1b_ref, k2_ref,
                k3_ref, w_ref, b_ref, pw_ref, pb_ref, out_ref, *, last):
    q = jnp.dot(og_ref[...], k1a_ref[...], preferred_element_type=jnp.float32)
    k1b = k1b_ref[...]
    k2 = k2_ref[...]
    k3 = k3_ref[...]
    acc = jnp.zeros((BN, D), jnp.float32)
    for k in range(K):
        pj_k = pj_ref[:, k * 8:(k + 1) * 8]
        h1 = jax.nn.gelu(q + jnp.dot(pj_k, k1b,
                                     preferred_element_type=jnp.float32))
        h2 = jax.nn.gelu(jnp.dot(h1, k2, preferred_element_type=jnp.float32))
        kap = jnp.dot(h2, k3, preferred_element_type=jnp.float32)
        acc = acc + kap * xj_ref[:, k * D:(k + 1) * D]
    msg = acc * (1.0 / K)
    xn = jax.nn.gelu(jnp.dot(msg, w_ref[...],
                             preferred_element_type=jnp.float32)
                     + b_ref[...] + x_ref[...])
    if last:
        out_ref[...] = jnp.dot(xn, pw_ref[...],
                               preferred_element_type=jnp.float32) + pb_ref[...]
    else:
        out_ref[...] = xn


def _full(shape):
    return pl.BlockSpec(shape, lambda b: (0,) * len(shape))


def _rows(width):
    return pl.BlockSpec((BN, width), lambda b: (b, 0))


def _tc_pre(inp_p, im128, igd128, ogd128, lift_W, lift_b):
    return pl.pallas_call(
        _pre_body,
        grid=(GRID,),
        in_specs=[_rows(D), _rows(D), _rows(D), _rows(D),
                  _full((D, D)), _full((1, D))],
        out_specs=[_rows(D), _rows(D), _rows(D)],
        out_shape=[jax.ShapeDtypeStruct((NPAD, D), jnp.float32),
                   jax.ShapeDtypeStruct((NPAD, D), jnp.float32),
                   jax.ShapeDtypeStruct((NPAD, D), jnp.float32)],
    )(inp_p, im128, igd128, ogd128, lift_W, lift_b)


def _tc_compact(posw):
    return pl.pallas_call(
        _compact_body,
        grid=(GRID,),
        in_specs=[_rows(K * D)],
        out_specs=_rows(K * 8),
        out_shape=jax.ShapeDtypeStruct((NPAD, K * 8), jnp.float32),
    )(posw)


def _tc_layer(og, pjv, xjv, x, k1a, k1b, k2, k3, w, b, pw, pb, last):
    return pl.pallas_call(
        functools.partial(_layer_body, last=last),
        grid=(GRID,),
        in_specs=[_rows(D), _rows(K * 8), _rows(K * D), _rows(D),
                  _full((D, H)), _full((8, H)), _full((H, H)), _full((H, D)),
                  _full((D, D)), _full((1, D)), _full((D, D)), _full((1, D))],
        out_specs=_rows(D),
        out_shape=jax.ShapeDtypeStruct((NPAD, D), jnp.float32),
    )(og, pjv, xjv, x, k1a, k1b, k2, k3, w, b, pw, pb)


def kernel(inp, out_grid_displacement, in_grid_displacement, neighbor_idx,
           initial_mesh, lift_W, lift_b, K1, K2, K3, Wl, bl, proj_W, proj_b):
    f32 = jnp.float32

    def pad_n(a, width):
        out = jnp.zeros((NPAD, width), f32)
        return out.at[:N, :a.shape[1]].set(a)

    inp_p = pad_n(inp[0], D)
    im128 = pad_n(initial_mesh, D)
    igd128 = pad_n(in_grid_displacement, D)
    ogd128 = pad_n(out_grid_displacement, D)

    idx_flat = jnp.zeros((NE,), jnp.int32)
    idx_flat = idx_flat.at[:N * K].set(neighbor_idx.reshape(-1))

    lb = lift_b.reshape(1, D)
    pb = proj_b.reshape(1, D)

    k1a = jnp.zeros((L, D, H), f32).at[:, :ND, :].set(K1[:, :ND, :])
    k1b = jnp.zeros((L, 8, H), f32).at[:, :ND, :].set(K1[:, ND:, :])

    x, ing, outg = _tc_pre(inp_p, im128, igd128, ogd128, lift_W, lb)

    posw = _sc_gather(ing, idx_flat, D, 640)
    pjv = _tc_compact(posw.reshape(NPAD, K * D))

    for i in range(L):
        xj = _sc_gather(x, idx_flat, D, 640)
        xjv = xj.reshape(NPAD, K * D)
        og = outg if i == L - 1 else ing
        x = _tc_layer(og, pjv, xjv, x, k1a[i], k1b[i], K2[i], K3[i],
                      Wl[i], bl[i].reshape(1, D), proj_W, pb, i == L - 1)

    return x[:N][None]


# speedup vs baseline: 1.0500x; 1.0500x over previous
import functools

import jax
import jax.numpy as jnp
from jax import lax
from jax.experimental import pallas as pl
from jax.experimental.pallas import tpu as pltpu
from jax.experimental.pallas import tpu_sc as plsc

N = 10000
K = 16
D = 128
ND = 3
L = 4
H = 64

NPAD = 10240
NE = NPAD * K

BN = 256
GRID = NPAD // BN

NW = 32
ROWS_W = NE // NW


NBUF = 3


def _sc_gather(table, idx, wrow, chunk):
    dt = table.dtype
    nch = ROWS_W // chunk
    mesh = plsc.VectorSubcoreMesh(core_axis_name="c", subcore_axis_name="s")

    @functools.partial(
        pl.kernel,
        mesh=mesh,
        out_type=jax.ShapeDtypeStruct((NE, wrow), dt),
        scratch_types=[
            pltpu.VMEM((ROWS_W,), jnp.int32),
        ] + [pltpu.VMEM((chunk, wrow), dt) for _ in range(NBUF)]
          + [pltpu.SemaphoreType.DMA for _ in range(2 * NBUF)],
    )
    def gather_k(table_hbm, idx_hbm, out_hbm, idx_v, *bufs_and_sems):
        bufs = bufs_and_sems[:NBUF]
        sem_g = bufs_and_sems[NBUF:2 * NBUF]
        sem_o = bufs_and_sems[2 * NBUF:]
        wid = lax.axis_index("s") * 2 + lax.axis_index("c")
        base = wid * ROWS_W
        pltpu.sync_copy(idx_hbm.at[pl.ds(base, ROWS_W)], idx_v)

        g_h, o_h = {}, {}

        def start_g(c):
            g_h[c] = pltpu.async_copy(
                table_hbm.at[idx_v.at[pl.ds(c * chunk, chunk)]],
                bufs[c % NBUF], sem_g[c % NBUF])

        def start_o(c):
            o_h[c] = pltpu.async_copy(
                bufs[c % NBUF], out_hbm.at[pl.ds(base + c * chunk, chunk)],
                sem_o[c % NBUF])

        for c in range(min(NBUF - 1, nch)):
            start_g(c)
        for c in range(nch):
            g_h[c].wait()
            start_o(c)
            nxt = c + NBUF - 1
            if nxt < nch:
                if nxt >= NBUF:
                    o_h[nxt - NBUF].wait()
                start_g(nxt)
        for c in range(max(0, nch - NBUF), nch):
            o_h[c].wait()

    return gather_k(table, idx)


def _pre_body(inp_ref, im_ref, igd_ref, ogd_ref, lw_ref, lb_ref,
              x0_ref, ing_ref, outg_ref):
    x = jnp.dot(inp_ref[...], lw_ref[...], preferred_element_type=jnp.float32)
    x0_ref[...] = jax.nn.gelu(x + lb_ref[...])
    im = im_ref[...]
    ing_ref[...] = im + igd_ref[...]
    outg_ref[...] = im + ogd_ref[...]


def _compact_body(pw_ref, pj_ref):
    for k in range(K):
        pj_ref[:, k * 8:(k + 1) * 8] = pw_ref[:, k * D:k * D + 8]


def _layer_core(og_ref, pj_ref, xj_ref, x_ref, k1a_ref, k1b_ref, k2_ref,
                k3_ref, w_ref, b_ref):
    q = jnp.dot(og_ref[...], k1a_ref[...], preferred_element_type=jnp.float32)
    k1b = k1b_ref[...]
    k2 = k2_ref[...]
    k3 = k3_ref[...]
    acc = jnp.zeros((BN, D), jnp.float32)
    for k in range(K):
        pj_k = pj_ref[:, k * 8:(k + 1) * 8]
        h1 = jax.nn.gelu(q + jnp.dot(pj_k, k1b,
                                     preferred_element_type=jnp.float32))
        h2 = jax.nn.gelu(jnp.dot(h1, k2, preferred_element_type=jnp.float32))
        kap = jnp.dot(h2, k3, preferred_element_type=jnp.float32)
        acc = acc + kap * xj_ref[:, k * D:(k + 1) * D].astype(jnp.float32)
    msg = acc * (1.0 / K)
    return jax.nn.gelu(jnp.dot(msg, w_ref[...],
                               preferred_element_type=jnp.float32)
                       + b_ref[...] + x_ref[...])


def _layer_body(og_ref, pj_ref, xj_ref, x_ref, k1a_ref, k1b_ref, k2_ref,
                k3_ref, w_ref, b_ref, out_ref):
    out_ref[...] = _layer_core(og_ref, pj_ref, xj_ref, x_ref, k1a_ref,
                               k1b_ref, k2_ref, k3_ref, w_ref, b_ref)


def _layer_body_last(og_ref, pj_ref, xj_ref, x_ref, k1a_ref, k1b_ref, k2_ref,
                     k3_ref, w_ref, b_ref, pw_ref, pb_ref, out_ref):
    xn = _layer_core(og_ref, pj_ref, xj_ref, x_ref, k1a_ref, k1b_ref,
                     k2_ref, k3_ref, w_ref, b_ref)
    out_ref[...] = jnp.dot(xn, pw_ref[...],
                           preferred_element_type=jnp.float32) + pb_ref[...]


def _full(shape):
    return pl.BlockSpec(shape, lambda b: (0,) * len(shape))


def _rows(width):
    return pl.BlockSpec((BN, width), lambda b: (b, 0))


def _tc_pre(inp_p, im128, igd128, ogd128, lift_W, lift_b):
    return pl.pallas_call(
        _pre_body,
        grid=(GRID,),
        in_specs=[_rows(D), _rows(D), _rows(D), _rows(D),
                  _full((D, D)), _full((1, D))],
        out_specs=[_rows(D), _rows(D), _rows(D)],
        out_shape=[jax.ShapeDtypeStruct((NPAD, D), jnp.float32),
                   jax.ShapeDtypeStruct((NPAD, D), jnp.float32),
                   jax.ShapeDtypeStruct((NPAD, D), jnp.float32)],
    )(inp_p, im128, igd128, ogd128, lift_W, lift_b)


def _tc_compact(posw):
    return pl.pallas_call(
        _compact_body,
        grid=(GRID,),
        in_specs=[_rows(K * D)],
        out_specs=_rows(K * 8),
        out_shape=jax.ShapeDtypeStruct((NPAD, K * 8), jnp.float32),
    )(posw)


def _tc_layer(og, pjv, xjv, x, k1a, k1b, k2, k3, w, b):
    return pl.pallas_call(
        _layer_body,
        grid=(GRID,),
        in_specs=[_rows(D), _rows(K * 8), _rows(K * D), _rows(D),
                  _full((D, H)), _full((8, H)), _full((H, H)), _full((H, D)),
                  _full((D, D)), _full((1, D))],
        out_specs=_rows(D),
        out_shape=jax.ShapeDtypeStruct((NPAD, D), jnp.float32),
    )(og, pjv, xjv, x, k1a, k1b, k2, k3, w, b)


def _tc_layer_last(og, pjv, xjv, x, k1a, k1b, k2, k3, w, b, pw, pb):
    return pl.pallas_call(
        _layer_body_last,
        grid=(GRID,),
        in_specs=[_rows(D), _rows(K * 8), _rows(K * D), _rows(D),
                  _full((D, H)), _full((8, H)), _full((H, H)), _full((H, D)),
                  _full((D, D)), _full((1, D)), _full((D, D)), _full((1, D))],
        out_specs=_rows(D),
        out_shape=jax.ShapeDtypeStruct((NPAD, D), jnp.float32),
    )(og, pjv, xjv, x, k1a, k1b, k2, k3, w, b, pw, pb)


def kernel(inp, out_grid_displacement, in_grid_displacement, neighbor_idx,
           initial_mesh, lift_W, lift_b, K1, K2, K3, Wl, bl, proj_W, proj_b):
    f32 = jnp.float32

    def pad_n(a, width):
        out = jnp.zeros((NPAD, width), f32)
        return out.at[:N, :a.shape[1]].set(a)

    inp_p = pad_n(inp[0], D)
    im128 = pad_n(initial_mesh, D)
    igd128 = pad_n(in_grid_displacement, D)
    ogd128 = pad_n(out_grid_displacement, D)

    idx_flat = jnp.zeros((NE,), jnp.int32)
    idx_flat = idx_flat.at[:N * K].set(neighbor_idx.reshape(-1))

    lb = lift_b.reshape(1, D)
    pb = proj_b.reshape(1, D)

    k1a = jnp.zeros((L, D, H), f32).at[:, :ND, :].set(K1[:, :ND, :])
    k1b = jnp.zeros((L, 8, H), f32).at[:, :ND, :].set(K1[:, ND:, :])

    x, ing, outg = _tc_pre(inp_p, im128, igd128, ogd128, lift_W, lb)

    posw = _sc_gather(ing, idx_flat, D, 256)
    pjv = _tc_compact(posw.reshape(NPAD, K * D))

    for i in range(L):
        xj = _sc_gather(x, idx_flat, D, 320)
        xjv = xj.reshape(NPAD, K * D)
        og = outg if i == L - 1 else ing
        if i == L - 1:
            x = _tc_layer_last(og, pjv, xjv, x, k1a[i], k1b[i], K2[i], K3[i],
                               Wl[i], bl[i].reshape(1, D), proj_W, pb)
        else:
            x = _tc_layer(og, pjv, xjv, x, k1a[i], k1b[i], K2[i], K3[i],
                          Wl[i], bl[i].reshape(1, D))

    return x[:N][None]


# speedup vs baseline: 1.0521x; 1.0020x over previous
import functools

import jax
import jax.numpy as jnp
from jax import lax
from jax.experimental import pallas as pl
from jax.experimental.pallas import tpu as pltpu
from jax.experimental.pallas import tpu_sc as plsc

N = 10000
K = 16
D = 128
ND = 3
L = 4
H = 64

NPAD = 10240
NE = NPAD * K

BN = 256
GRID = NPAD // BN

NW = 32
ROWS_W = NE // NW


def _sc_gather(table, idx, wrow, chunk, nbuf):
    dt = table.dtype
    nch = ROWS_W // chunk
    mesh = plsc.VectorSubcoreMesh(core_axis_name="c", subcore_axis_name="s")

    @functools.partial(
        pl.kernel,
        mesh=mesh,
        out_type=jax.ShapeDtypeStruct((NE, wrow), dt),
        scratch_types=[
            pltpu.VMEM((ROWS_W,), jnp.int32),
        ] + [pltpu.VMEM((chunk, wrow), dt) for _ in range(nbuf)]
          + [pltpu.SemaphoreType.DMA for _ in range(2 * nbuf)],
    )
    def gather_k(table_hbm, idx_hbm, out_hbm, idx_v, *bufs_and_sems):
        bufs = bufs_and_sems[:nbuf]
        sem_g = bufs_and_sems[nbuf:2 * nbuf]
        sem_o = bufs_and_sems[2 * nbuf:]
        wid = lax.axis_index("s") * 2 + lax.axis_index("c")
        base = wid * ROWS_W
        pltpu.sync_copy(idx_hbm.at[pl.ds(base, ROWS_W)], idx_v)

        g_h, o_h = {}, {}

        def start_g(c):
            g_h[c] = pltpu.async_copy(
                table_hbm.at[idx_v.at[pl.ds(c * chunk, chunk)]],
                bufs[c % nbuf], sem_g[c % nbuf])

        def start_o(c):
            o_h[c] = pltpu.async_copy(
                bufs[c % nbuf], out_hbm.at[pl.ds(base + c * chunk, chunk)],
                sem_o[c % nbuf])

        for c in range(min(nbuf - 1, nch)):
            start_g(c)
        for c in range(nch):
            g_h[c].wait()
            start_o(c)
            nxt = c + nbuf - 1
            if nxt < nch:
                if nxt >= nbuf:
                    o_h[nxt - nbuf].wait()
                start_g(nxt)
        for c in range(max(0, nch - nbuf), nch):
            o_h[c].wait()

    return gather_k(table, idx)


def _pre_body(inp_ref, im_ref, igd_ref, ogd_ref, lw_ref, lb_ref,
              x0_ref, ing_ref, outg_ref):
    x = jnp.dot(inp_ref[...], lw_ref[...], preferred_element_type=jnp.float32)
    x0_ref[...] = jax.nn.gelu(x + lb_ref[...])
    im = im_ref[...]
    ing_ref[...] = im + igd_ref[...]
    outg_ref[...] = im + ogd_ref[...]


def _compact_body(pw_ref, pj_ref):
    for k in range(K):
        pj_ref[:, k * 8:(k + 1) * 8] = pw_ref[:, k * D:k * D + 8]


def _layer_core(og_ref, pj_ref, xj_ref, x_ref, k1a_ref, k1b_ref, k2_ref,
                k3_ref, w_ref, b_ref):
    q = jnp.dot(og_ref[...], k1a_ref[...], preferred_element_type=jnp.float32)
    k1b = k1b_ref[...]
    k2 = k2_ref[...]
    k3 = k3_ref[...]
    acc = jnp.zeros((BN, D), jnp.float32)
    for k in range(K):
        pj_k = pj_ref[:, k * 8:(k + 1) * 8]
        h1 = jax.nn.gelu(q + jnp.dot(pj_k, k1b,
                                     preferred_element_type=jnp.float32))
        h2 = jax.nn.gelu(jnp.dot(h1, k2, preferred_element_type=jnp.float32))
        kap = jnp.dot(h2, k3, preferred_element_type=jnp.float32)
        acc = acc + kap * xj_ref[:, k * D:(k + 1) * D].astype(jnp.float32)
    msg = acc * (1.0 / K)
    return jax.nn.gelu(jnp.dot(msg, w_ref[...],
                               preferred_element_type=jnp.float32)
                       + b_ref[...] + x_ref[...])


def _layer_body(og_ref, pj_ref, xj_ref, x_ref, k1a_ref, k1b_ref, k2_ref,
                k3_ref, w_ref, b_ref, out_ref):
    out_ref[...] = _layer_core(og_ref, pj_ref, xj_ref, x_ref, k1a_ref,
                               k1b_ref, k2_ref, k3_ref, w_ref, b_ref)


def _layer_body_last(og_ref, pj_ref, xj_ref, x_ref, k1a_ref, k1b_ref, k2_ref,
                     k3_ref, w_ref, b_ref, pw_ref, pb_ref, out_ref):
    xn = _layer_core(og_ref, pj_ref, xj_ref, x_ref, k1a_ref, k1b_ref,
                     k2_ref, k3_ref, w_ref, b_ref)
    out_ref[...] = jnp.dot(xn, pw_ref[...],
                           preferred_element_type=jnp.float32) + pb_ref[...]


def _full(shape):
    return pl.BlockSpec(shape, lambda b: (0,) * len(shape))


def _rows(width):
    return pl.BlockSpec((BN, width), lambda b: (b, 0))


def _tc_pre(inp_p, im128, igd128, ogd128, lift_W, lift_b):
    return pl.pallas_call(
        _pre_body,
        grid=(GRID,),
        in_specs=[_rows(D), _rows(D), _rows(D), _rows(D),
                  _full((D, D)), _full((1, D))],
        out_specs=[_rows(D), _rows(D), _rows(D)],
        out_shape=[jax.ShapeDtypeStruct((NPAD, D), jnp.float32),
                   jax.ShapeDtypeStruct((NPAD, D), jnp.float32),
                   jax.ShapeDtypeStruct((NPAD, D), jnp.float32)],
    )(inp_p, im128, igd128, ogd128, lift_W, lift_b)


def _tc_compact(posw):
    return pl.pallas_call(
        _compact_body,
        grid=(GRID,),
        in_specs=[_rows(K * D)],
        out_specs=_rows(K * 8),
        out_shape=jax.ShapeDtypeStruct((NPAD, K * 8), jnp.float32),
    )(posw)


def _tc_layer(og, pjv, xjv, x, k1a, k1b, k2, k3, w, b):
    return pl.pallas_call(
        _layer_body,
        grid=(GRID,),
        in_specs=[_rows(D), _rows(K * 8), _rows(K * D), _rows(D),
                  _full((D, H)), _full((8, H)), _full((H, H)), _full((H, D)),
                  _full((D, D)), _full((1, D))],
        out_specs=_rows(D),
        out_shape=jax.ShapeDtypeStruct((NPAD, D), jnp.float32),
    )(og, pjv, xjv, x, k1a, k1b, k2, k3, w, b)


def _tc_layer_last(og, pjv, xjv, x, k1a, k1b, k2, k3, w, b, pw, pb):
    return pl.pallas_call(
        _layer_body_last,
        grid=(GRID,),
        in_specs=[_rows(D), _rows(K * 8), _rows(K * D), _rows(D),
                  _full((D, H)), _full((8, H)), _full((H, H)), _full((H, D)),
                  _full((D, D)), _full((1, D)), _full((D, D)), _full((1, D))],
        out_specs=_rows(D),
        out_shape=jax.ShapeDtypeStruct((NPAD, D), jnp.float32),
    )(og, pjv, xjv, x, k1a, k1b, k2, k3, w, b, pw, pb)


def kernel(inp, out_grid_displacement, in_grid_displacement, neighbor_idx,
           initial_mesh, lift_W, lift_b, K1, K2, K3, Wl, bl, proj_W, proj_b):
    f32 = jnp.float32

    def pad_n(a, width):
        out = jnp.zeros((NPAD, width), f32)
        return out.at[:N, :a.shape[1]].set(a)

    inp_p = pad_n(inp[0], D)
    im128 = pad_n(initial_mesh, D)
    igd128 = pad_n(in_grid_displacement, D)
    ogd128 = pad_n(out_grid_displacement, D)

    idx_flat = jnp.zeros((NE,), jnp.int32)
    idx_flat = idx_flat.at[:N * K].set(neighbor_idx.reshape(-1))

    lb = lift_b.reshape(1, D)
    pb = proj_b.reshape(1, D)

    k1a = jnp.zeros((L, D, H), f32).at[:, :ND, :].set(K1[:, :ND, :])
    k1b = jnp.zeros((L, 8, H), f32).at[:, :ND, :].set(K1[:, ND:, :])

    x, ing, outg = _tc_pre(inp_p, im128, igd128, ogd128, lift_W, lb)

    posw = _sc_gather(ing, idx_flat, D, 160, 6)
    pjv = _tc_compact(posw.reshape(NPAD, K * D))

    for i in range(L):
        xj = _sc_gather(x, idx_flat, D, 160, 6)
        xjv = xj.reshape(NPAD, K * D)
        og = outg if i == L - 1 else ing
        if i == L - 1:
            x = _tc_layer_last(og, pjv, xjv, x, k1a[i], k1b[i], K2[i], K3[i],
                               Wl[i], bl[i].reshape(1, D), proj_W, pb)
        else:
            x = _tc_layer(og, pjv, xjv, x, k1a[i], k1b[i], K2[i], K3[i],
                          Wl[i], bl[i].reshape(1, D))

    return x[:N][None]


# speedup vs baseline: 1.8436x; 1.7523x over previous
import functools

import jax
import jax.numpy as jnp
from jax import lax
from jax.experimental import pallas as pl
from jax.experimental.pallas import tpu as pltpu
from jax.experimental.pallas import tpu_sc as plsc

N = 10000
K = 16
D = 128
ND = 3
L = 4
H = 64

NPAD = 10240
NE = NPAD * K

BN = 256
GRID = NPAD // BN

NW = 32
ROWS_W = NE // NW


def _sc_gather(table, idx, wrow, chunk, nbuf):
    dt = table.dtype
    nch = ROWS_W // chunk
    mesh = plsc.VectorSubcoreMesh(core_axis_name="c", subcore_axis_name="s")

    @functools.partial(
        pl.kernel,
        mesh=mesh,
        out_type=jax.ShapeDtypeStruct((NE, wrow), dt),
        scratch_types=[
            pltpu.VMEM((ROWS_W,), jnp.int32),
        ] + [pltpu.VMEM((chunk, wrow), dt) for _ in range(nbuf)]
          + [pltpu.SemaphoreType.DMA for _ in range(2 * nbuf)],
    )
    def gather_k(table_hbm, idx_hbm, out_hbm, idx_v, *bufs_and_sems):
        bufs = bufs_and_sems[:nbuf]
        sem_g = bufs_and_sems[nbuf:2 * nbuf]
        sem_o = bufs_and_sems[2 * nbuf:]
        wid = lax.axis_index("s") * 2 + lax.axis_index("c")
        base = wid * ROWS_W
        pltpu.sync_copy(idx_hbm.at[pl.ds(base, ROWS_W)], idx_v)

        g_h, o_h = {}, {}

        def start_g(c):
            g_h[c] = pltpu.async_copy(
                table_hbm.at[idx_v.at[pl.ds(c * chunk, chunk)]],
                bufs[c % nbuf], sem_g[c % nbuf])

        def start_o(c):
            o_h[c] = pltpu.async_copy(
                bufs[c % nbuf], out_hbm.at[pl.ds(base + c * chunk, chunk)],
                sem_o[c % nbuf])

        for c in range(min(nbuf - 1, nch)):
            start_g(c)
        for c in range(nch):
            g_h[c].wait()
            start_o(c)
            nxt = c + nbuf - 1
            if nxt < nch:
                if nxt >= nbuf:
                    o_h[nxt - nbuf].wait()
                start_g(nxt)
        for c in range(max(0, nch - nbuf), nch):
            o_h[c].wait()

    return gather_k(table, idx)


def _sc_gather_spmem(table, idx, wrow, chunk, nbuf):
    dt = table.dtype
    nch = ROWS_W // chunk
    stage = NPAD // 16
    mesh = plsc.VectorSubcoreMesh(core_axis_name="c", subcore_axis_name="s")

    @functools.partial(
        pl.kernel,
        mesh=mesh,
        out_type=jax.ShapeDtypeStruct((NE, wrow), dt),
        scratch_types=[
            pltpu.VMEM_SHARED((NPAD, wrow), dt),
            pltpu.VMEM((ROWS_W,), jnp.int32),
        ] + [pltpu.VMEM((chunk, wrow), dt) for _ in range(nbuf)]
          + [pltpu.SemaphoreType.DMA for _ in range(2 * nbuf)],
    )
    def gather_k(table_hbm, idx_hbm, out_hbm, tab_s, idx_v, *bufs_and_sems):
        bufs = bufs_and_sems[:nbuf]
        sem_g = bufs_and_sems[nbuf:2 * nbuf]
        sem_o = bufs_and_sems[2 * nbuf:]
        sid = lax.axis_index("s")
        wid = sid * 2 + lax.axis_index("c")
        base = wid * ROWS_W
        pltpu.sync_copy(table_hbm.at[pl.ds(sid * stage, stage)],
                        tab_s.at[pl.ds(sid * stage, stage)])
        pltpu.sync_copy(idx_hbm.at[pl.ds(base, ROWS_W)], idx_v)
        plsc.subcore_barrier()

        g_h, o_h = {}, {}

        def start_g(c):
            g_h[c] = pltpu.async_copy(
                tab_s.at[idx_v.at[pl.ds(c * chunk, chunk)]],
                bufs[c % nbuf], sem_g[c % nbuf])

        def start_o(c):
            o_h[c] = pltpu.async_copy(
                bufs[c % nbuf], out_hbm.at[pl.ds(base + c * chunk, chunk)],
                sem_o[c % nbuf])

        for c in range(min(nbuf - 1, nch)):
            start_g(c)
        for c in range(nch):
            g_h[c].wait()
            start_o(c)
            nxt = c + nbuf - 1
            if nxt < nch:
                if nxt >= nbuf:
                    o_h[nxt - nbuf].wait()
                start_g(nxt)
        for c in range(max(0, nch - nbuf), nch):
            o_h[c].wait()

    return gather_k(table, idx)


def _pre_body(inp_ref, im_ref, igd_ref, ogd_ref, lw_ref, lb_ref,
              x0_ref, ing_ref, outg_ref):
    x = jnp.dot(inp_ref[...], lw_ref[...], preferred_element_type=jnp.float32)
    x0_ref[...] = jax.nn.gelu(x + lb_ref[...])
    im = im_ref[...]
    ing_ref[...] = im + igd_ref[...]
    outg_ref[...] = im + ogd_ref[...]


def _compact_body(pw_ref, pj_ref):
    for k in range(K):
        pj_ref[:, k * 8:(k + 1) * 8] = pw_ref[:, k * D:k * D + 8]


def _layer_core(og_ref, pj_ref, xj_ref, x_ref, k1a_ref, k1b_ref, k2_ref,
                k3_ref, w_ref, b_ref):
    q = jnp.dot(og_ref[...], k1a_ref[...], preferred_element_type=jnp.float32)
    k1b = k1b_ref[...]
    k2 = k2_ref[...]
    k3 = k3_ref[...]
    acc = jnp.zeros((BN, D), jnp.float32)
    for k in range(K):
        pj_k = pj_ref[:, k * 8:(k + 1) * 8]
        h1 = jax.nn.gelu(q + jnp.dot(pj_k, k1b,
                                     preferred_element_type=jnp.float32))
        h2 = jax.nn.gelu(jnp.dot(h1, k2, preferred_element_type=jnp.float32))
        kap = jnp.dot(h2, k3, preferred_element_type=jnp.float32)
        acc = acc + kap * xj_ref[:, k * D:(k + 1) * D].astype(jnp.float32)
    msg = acc * (1.0 / K)
    return jax.nn.gelu(jnp.dot(msg, w_ref[...],
                               preferred_element_type=jnp.float32)
                       + b_ref[...] + x_ref[...])


def _layer_body(og_ref, pj_ref, xj_ref, x_ref, k1a_ref, k1b_ref, k2_ref,
                k3_ref, w_ref, b_ref, out_ref):
    out_ref[...] = _layer_core(og_ref, pj_ref, xj_ref, x_ref, k1a_ref,
                               k1b_ref, k2_ref, k3_ref, w_ref, b_ref)


def _layer_body_last(og_ref, pj_ref, xj_ref, x_ref, k1a_ref, k1b_ref, k2_ref,
                     k3_ref, w_ref, b_ref, pw_ref, pb_ref, out_ref):
    xn = _layer_core(og_ref, pj_ref, xj_ref, x_ref, k1a_ref, k1b_ref,
                     k2_ref, k3_ref, w_ref, b_ref)
    out_ref[...] = jnp.dot(xn, pw_ref[...],
                           preferred_element_type=jnp.float32) + pb_ref[...]


def _full(shape):
    return pl.BlockSpec(shape, lambda b: (0,) * len(shape))


def _rows(width):
    return pl.BlockSpec((BN, width), lambda b: (b, 0))


def _tc_pre(inp_p, im128, igd128, ogd128, lift_W, lift_b):
    return pl.pallas_call(
        _pre_body,
        grid=(GRID,),
        in_specs=[_rows(D), _rows(D), _rows(D), _rows(D),
                  _full((D, D)), _full((1, D))],
        out_specs=[_rows(D), _rows(D), _rows(D)],
        out_shape=[jax.ShapeDtypeStruct((NPAD, D), jnp.float32),
                   jax.ShapeDtypeStruct((NPAD, D), jnp.float32),
                   jax.ShapeDtypeStruct((NPAD, D), jnp.float32)],
    )(inp_p, im128, igd128, ogd128, lift_W, lift_b)


def _tc_compact(posw):
    return pl.pallas_call(
        _compact_body,
        grid=(GRID,),
        in_specs=[_rows(K * D)],
        out_specs=_rows(K * 8),
        out_shape=jax.ShapeDtypeStruct((NPAD, K * 8), jnp.float32),
    )(posw)


def _tc_layer(og, pjv, xjv, x, k1a, k1b, k2, k3, w, b):
    return pl.pallas_call(
        _layer_body,
        grid=(GRID,),
        in_specs=[_rows(D), _rows(K * 8), _rows(K * D), _rows(D),
                  _full((D, H)), _full((8, H)), _full((H, H)), _full((H, D)),
                  _full((D, D)), _full((1, D))],
        out_specs=_rows(D),
        out_shape=jax.ShapeDtypeStruct((NPAD, D), jnp.float32),
    )(og, pjv, xjv, x, k1a, k1b, k2, k3, w, b)


def _tc_layer_last(og, pjv, xjv, x, k1a, k1b, k2, k3, w, b, pw, pb):
    return pl.pallas_call(
        _layer_body_last,
        grid=(GRID,),
        in_specs=[_rows(D), _rows(K * 8), _rows(K * D), _rows(D),
                  _full((D, H)), _full((8, H)), _full((H, H)), _full((H, D)),
                  _full((D, D)), _full((1, D)), _full((D, D)), _full((1, D))],
        out_specs=_rows(D),
        out_shape=jax.ShapeDtypeStruct((NPAD, D), jnp.float32),
    )(og, pjv, xjv, x, k1a, k1b, k2, k3, w, b, pw, pb)


def kernel(inp, out_grid_displacement, in_grid_displacement, neighbor_idx,
           initial_mesh, lift_W, lift_b, K1, K2, K3, Wl, bl, proj_W, proj_b):
    f32 = jnp.float32

    def pad_n(a, width):
        out = jnp.zeros((NPAD, width), f32)
        return out.at[:N, :a.shape[1]].set(a)

    inp_p = pad_n(inp[0], D)
    im128 = pad_n(initial_mesh, D)
    igd128 = pad_n(in_grid_displacement, D)
    ogd128 = pad_n(out_grid_displacement, D)

    idx_flat = jnp.zeros((NE,), jnp.int32)
    idx_flat = idx_flat.at[:N * K].set(neighbor_idx.reshape(-1))

    lb = lift_b.reshape(1, D)
    pb = proj_b.reshape(1, D)

    k1a = jnp.zeros((L, D, H), f32).at[:, :ND, :].set(K1[:, :ND, :])
    k1b = jnp.zeros((L, 8, H), f32).at[:, :ND, :].set(K1[:, ND:, :])

    x, ing, outg = _tc_pre(inp_p, im128, igd128, ogd128, lift_W, lb)

    posw = _sc_gather_spmem(ing, idx_flat, D, 160, 2)
    pjv = _tc_compact(posw.reshape(NPAD, K * D))

    for i in range(L):
        xj = _sc_gather_spmem(x, idx_flat, D, 160, 2)
        xjv = xj.reshape(NPAD, K * D)
        og = outg if i == L - 1 else ing
        if i == L - 1:
            x = _tc_layer_last(og, pjv, xjv, x, k1a[i], k1b[i], K2[i], K3[i],
                               Wl[i], bl[i].reshape(1, D), proj_W, pb)
        else:
            x = _tc_layer(og, pjv, xjv, x, k1a[i], k1b[i], K2[i], K3[i],
                          Wl[i], bl[i].reshape(1, D))

    return x[:N][None]


# speedup vs baseline: 2.5282x; 1.3713x over previous
import functools

import jax
import jax.numpy as jnp
from jax import lax
from jax.experimental import pallas as pl
from jax.experimental.pallas import tpu as pltpu
from jax.experimental.pallas import tpu_sc as plsc

N = 10000
K = 16
D = 128
ND = 3
L = 4
H = 64

NPAD = 10240
NE = NPAD * K

BN = 512
GRID = NPAD // BN

NW = 32
ROWS_W = NE // NW


def _sc_gather(table, idx, wrow, chunk, nbuf):
    dt = table.dtype
    nch = ROWS_W // chunk
    mesh = plsc.VectorSubcoreMesh(core_axis_name="c", subcore_axis_name="s")

    @functools.partial(
        pl.kernel,
        mesh=mesh,
        out_type=jax.ShapeDtypeStruct((NE, wrow), dt),
        scratch_types=[
            pltpu.VMEM((ROWS_W,), jnp.int32),
        ] + [pltpu.VMEM((chunk, wrow), dt) for _ in range(nbuf)]
          + [pltpu.SemaphoreType.DMA for _ in range(2 * nbuf)],
    )
    def gather_k(table_hbm, idx_hbm, out_hbm, idx_v, *bufs_and_sems):
        bufs = bufs_and_sems[:nbuf]
        sem_g = bufs_and_sems[nbuf:2 * nbuf]
        sem_o = bufs_and_sems[2 * nbuf:]
        wid = lax.axis_index("s") * 2 + lax.axis_index("c")
        base = wid * ROWS_W
        pltpu.sync_copy(idx_hbm.at[pl.ds(base, ROWS_W)], idx_v)

        g_h, o_h = {}, {}

        def start_g(c):
            g_h[c] = pltpu.async_copy(
                table_hbm.at[idx_v.at[pl.ds(c * chunk, chunk)]],
                bufs[c % nbuf], sem_g[c % nbuf])

        def start_o(c):
            o_h[c] = pltpu.async_copy(
                bufs[c % nbuf], out_hbm.at[pl.ds(base + c * chunk, chunk)],
                sem_o[c % nbuf])

        for c in range(min(nbuf - 1, nch)):
            start_g(c)
        for c in range(nch):
            g_h[c].wait()
            start_o(c)
            nxt = c + nbuf - 1
            if nxt < nch:
                if nxt >= nbuf:
                    o_h[nxt - nbuf].wait()
                start_g(nxt)
        for c in range(max(0, nch - nbuf), nch):
            o_h[c].wait()

    return gather_k(table, idx)


def _sc_gather_spmem(table, idx, wrow, chunk, nbuf):
    dt = table.dtype
    nch = ROWS_W // chunk
    stage = NPAD // 16
    mesh = plsc.VectorSubcoreMesh(core_axis_name="c", subcore_axis_name="s")

    @functools.partial(
        pl.kernel,
        mesh=mesh,
        out_type=jax.ShapeDtypeStruct((NE, wrow), dt),
        scratch_types=[
            pltpu.VMEM_SHARED((NPAD, wrow), dt),
            pltpu.VMEM((ROWS_W,), jnp.int32),
        ] + [pltpu.VMEM((chunk, wrow), dt) for _ in range(nbuf)]
          + [pltpu.SemaphoreType.DMA for _ in range(2 * nbuf)],
    )
    def gather_k(table_hbm, idx_hbm, out_hbm, tab_s, idx_v, *bufs_and_sems):
        bufs = bufs_and_sems[:nbuf]
        sem_g = bufs_and_sems[nbuf:2 * nbuf]
        sem_o = bufs_and_sems[2 * nbuf:]
        sid = lax.axis_index("s")
        wid = sid * 2 + lax.axis_index("c")
        base = wid * ROWS_W
        pltpu.sync_copy(table_hbm.at[pl.ds(sid * stage, stage)],
                        tab_s.at[pl.ds(sid * stage, stage)])
        pltpu.sync_copy(idx_hbm.at[pl.ds(base, ROWS_W)], idx_v)
        plsc.subcore_barrier()

        g_h, o_h = {}, {}

        def start_g(c):
            g_h[c] = pltpu.async_copy(
                tab_s.at[idx_v.at[pl.ds(c * chunk, chunk)]],
                bufs[c % nbuf], sem_g[c % nbuf])

        def start_o(c):
            o_h[c] = pltpu.async_copy(
                bufs[c % nbuf], out_hbm.at[pl.ds(base + c * chunk, chunk)],
                sem_o[c % nbuf])

        for c in range(min(nbuf - 1, nch)):
            start_g(c)
        for c in range(nch):
            g_h[c].wait()
            start_o(c)
            nxt = c + nbuf - 1
            if nxt < nch:
                if nxt >= nbuf:
                    o_h[nxt - nbuf].wait()
                start_g(nxt)
        for c in range(max(0, nch - nbuf), nch):
            o_h[c].wait()

    return gather_k(table, idx)


def _pre_body(inp_ref, im_ref, igd_ref, ogd_ref, lw_ref, lb_ref,
              x0_ref, ing_ref, outg_ref):
    x = jnp.dot(inp_ref[...], lw_ref[...], preferred_element_type=jnp.float32)
    x0_ref[...] = jax.nn.gelu(x + lb_ref[...])
    im = im_ref[...]
    ing_ref[...] = im + igd_ref[...]
    outg_ref[...] = im + ogd_ref[...]


def _compact_body(pw_ref, pj_ref):
    for k in range(K):
        pj_ref[:, k * 8:(k + 1) * 8] = pw_ref[:, k * D:k * D + 8]


def _layer_core(og_ref, pj_ref, xj_ref, x_ref, k1a_ref, k1b_ref, k2_ref,
                k3_ref, w_ref, b_ref):
    qp = jnp.dot(og_ref[...], k1a_ref[...], preferred_element_type=jnp.float32)
    k1b = k1b_ref[...]
    k2 = k2_ref[...]
    k3 = k3_ref[...]
    acc = jnp.zeros((BN, D), jnp.float32)
    for kk in range(K // 2):
        pj_p = pj_ref[:, kk * 16:(kk + 1) * 16]
        h1 = jax.nn.gelu(qp + jnp.dot(pj_p, k1b,
                                      preferred_element_type=jnp.float32))
        h2 = jax.nn.gelu(jnp.dot(h1, k2, preferred_element_type=jnp.float32))
        kap2 = jnp.dot(h2, k3, preferred_element_type=jnp.float32)
        acc = (acc
               + kap2[:, :D] * xj_ref[:, (2 * kk) * D:(2 * kk + 1) * D]
               + kap2[:, D:] * xj_ref[:, (2 * kk + 1) * D:(2 * kk + 2) * D])
    msg = acc * (1.0 / K)
    return jax.nn.gelu(jnp.dot(msg, w_ref[...],
                               preferred_element_type=jnp.float32)
                       + b_ref[...] + x_ref[...])


def _layer_body(og_ref, pj_ref, xj_ref, x_ref, k1a_ref, k1b_ref, k2_ref,
                k3_ref, w_ref, b_ref, out_ref):
    out_ref[...] = _layer_core(og_ref, pj_ref, xj_ref, x_ref, k1a_ref,
                               k1b_ref, k2_ref, k3_ref, w_ref, b_ref)


def _layer_body_last(og_ref, pj_ref, xj_ref, x_ref, k1a_ref, k1b_ref, k2_ref,
                     k3_ref, w_ref, b_ref, pw_ref, pb_ref, out_ref):
    xn = _layer_core(og_ref, pj_ref, xj_ref, x_ref, k1a_ref, k1b_ref,
                     k2_ref, k3_ref, w_ref, b_ref)
    out_ref[...] = jnp.dot(xn, pw_ref[...],
                           preferred_element_type=jnp.float32) + pb_ref[...]


def _full(shape):
    return pl.BlockSpec(shape, lambda b: (0,) * len(shape))


def _rows(width):
    return pl.BlockSpec((BN, width), lambda b: (b, 0))


def _tc_pre(inp_p, im128, igd128, ogd128, lift_W, lift_b):
    return pl.pallas_call(
        _pre_body,
        grid=(GRID,),
        in_specs=[_rows(D), _rows(D), _rows(D), _rows(D),
                  _full((D, D)), _full((1, D))],
        out_specs=[_rows(D), _rows(D), _rows(D)],
        out_shape=[jax.ShapeDtypeStruct((NPAD, D), jnp.float32),
                   jax.ShapeDtypeStruct((NPAD, D), jnp.float32),
                   jax.ShapeDtypeStruct((NPAD, D), jnp.float32)],
    )(inp_p, im128, igd128, ogd128, lift_W, lift_b)


def _tc_compact(posw):
    return pl.pallas_call(
        _compact_body,
        grid=(GRID,),
        in_specs=[_rows(K * D)],
        out_specs=_rows(K * 8),
        out_shape=jax.ShapeDtypeStruct((NPAD, K * 8), jnp.float32),
    )(posw)


def _tc_layer(og, pjv, xjv, x, k1a, k1b, k2, k3, w, b):
    return pl.pallas_call(
        _layer_body,
        grid=(GRID,),
        in_specs=[_rows(D), _rows(K * 8), _rows(K * D), _rows(D),
                  _full((D, D)), _full((16, D)), _full((D, D)), _full((D, 2 * D)),
                  _full((D, D)), _full((1, D))],
        out_specs=_rows(D),
        out_shape=jax.ShapeDtypeStruct((NPAD, D), jnp.float32),
    )(og, pjv, xjv, x, k1a, k1b, k2, k3, w, b)


def _tc_layer_last(og, pjv, xjv, x, k1a, k1b, k2, k3, w, b, pw, pb):
    return pl.pallas_call(
        _layer_body_last,
        grid=(GRID,),
        in_specs=[_rows(D), _rows(K * 8), _rows(K * D), _rows(D),
                  _full((D, D)), _full((16, D)), _full((D, D)), _full((D, 2 * D)),
                  _full((D, D)), _full((1, D)), _full((D, D)), _full((1, D))],
        out_specs=_rows(D),
        out_shape=jax.ShapeDtypeStruct((NPAD, D), jnp.float32),
    )(og, pjv, xjv, x, k1a, k1b, k2, k3, w, b, pw, pb)


def kernel(inp, out_grid_displacement, in_grid_displacement, neighbor_idx,
           initial_mesh, lift_W, lift_b, K1, K2, K3, Wl, bl, proj_W, proj_b):
    f32 = jnp.float32

    def pad_n(a, width):
        out = jnp.zeros((NPAD, width), f32)
        return out.at[:N, :a.shape[1]].set(a)

    inp_p = pad_n(inp[0], D)
    im128 = pad_n(initial_mesh, D)
    igd128 = pad_n(in_grid_displacement, D)
    ogd128 = pad_n(out_grid_displacement, D)

    idx_flat = jnp.zeros((NE,), jnp.int32)
    idx_flat = idx_flat.at[:N * K].set(neighbor_idx.reshape(-1))

    lb = lift_b.reshape(1, D)
    pb = proj_b.reshape(1, D)

    k1a_ = jnp.zeros((L, D, H), f32).at[:, :ND, :].set(K1[:, :ND, :])
    k1a = jnp.concatenate([k1a_, k1a_], axis=2)
    k1b_ = jnp.zeros((L, 8, H), f32).at[:, :ND, :].set(K1[:, ND:, :])
    k1b = jnp.zeros((L, 16, D), f32)
    k1b = k1b.at[:, :8, :H].set(k1b_).at[:, 8:, H:].set(k1b_)
    k2p = jnp.zeros((L, D, D), f32)
    k2p = k2p.at[:, :H, :H].set(K2).at[:, H:, H:].set(K2)
    k3p = jnp.zeros((L, D, 2 * D), f32)
    k3p = k3p.at[:, :H, :D].set(K3).at[:, H:, D:].set(K3)

    x, ing, outg = _tc_pre(inp_p, im128, igd128, ogd128, lift_W, lb)

    posw = _sc_gather_spmem(ing, idx_flat, D, 160, 2)
    pjv = _tc_compact(posw.reshape(NPAD, K * D))

    for i in range(L):
        xj = _sc_gather_spmem(x, idx_flat, D, 160, 2)
        xjv = xj.reshape(NPAD, K * D)
        og = outg if i == L - 1 else ing
        if i == L - 1:
            x = _tc_layer_last(og, pjv, xjv, x, k1a[i], k1b[i], k2p[i], k3p[i],
                               Wl[i], bl[i].reshape(1, D), proj_W, pb)
        else:
            x = _tc_layer(og, pjv, xjv, x, k1a[i], k1b[i], k2p[i], k3p[i],
                          Wl[i], bl[i].reshape(1, D))

    return x[:N][None]


# speedup vs baseline: 2.5284x; 1.0001x over previous
import functools

import jax
import jax.numpy as jnp
from jax import lax
from jax.experimental import pallas as pl
from jax.experimental.pallas import tpu as pltpu
from jax.experimental.pallas import tpu_sc as plsc

N = 10000
K = 16
D = 128
ND = 3
L = 4
H = 64

NPAD = 10240
NE = NPAD * K

BN = 512
GRID = NPAD // BN

NW = 32
ROWS_W = NE // NW


def _sc_gather(table, idx, wrow, chunk, nbuf):
    dt = table.dtype
    nch = ROWS_W // chunk
    mesh = plsc.VectorSubcoreMesh(core_axis_name="c", subcore_axis_name="s")

    @functools.partial(
        pl.kernel,
        mesh=mesh,
        out_type=jax.ShapeDtypeStruct((NE, wrow), dt),
        scratch_types=[
            pltpu.VMEM((ROWS_W,), jnp.int32),
        ] + [pltpu.VMEM((chunk, wrow), dt) for _ in range(nbuf)]
          + [pltpu.SemaphoreType.DMA for _ in range(2 * nbuf)],
    )
    def gather_k(table_hbm, idx_hbm, out_hbm, idx_v, *bufs_and_sems):
        bufs = bufs_and_sems[:nbuf]
        sem_g = bufs_and_sems[nbuf:2 * nbuf]
        sem_o = bufs_and_sems[2 * nbuf:]
        wid = lax.axis_index("s") * 2 + lax.axis_index("c")
        base = wid * ROWS_W
        pltpu.sync_copy(idx_hbm.at[pl.ds(base, ROWS_W)], idx_v)

        g_h, o_h = {}, {}

        def start_g(c):
            g_h[c] = pltpu.async_copy(
                table_hbm.at[idx_v.at[pl.ds(c * chunk, chunk)]],
                bufs[c % nbuf], sem_g[c % nbuf])

        def start_o(c):
            o_h[c] = pltpu.async_copy(
                bufs[c % nbuf], out_hbm.at[pl.ds(base + c * chunk, chunk)],
                sem_o[c % nbuf])

        for c in range(min(nbuf - 1, nch)):
            start_g(c)
        for c in range(nch):
            g_h[c].wait()
            start_o(c)
            nxt = c + nbuf - 1
            if nxt < nch:
                if nxt >= nbuf:
                    o_h[nxt - nbuf].wait()
                start_g(nxt)
        for c in range(max(0, nch - nbuf), nch):
            o_h[c].wait()

    return gather_k(table, idx)


def _sc_gather_spmem(table, idx, wrow, chunk, nbuf):
    dt = table.dtype
    nch = ROWS_W // chunk
    stage = NPAD // 16
    mesh = plsc.VectorSubcoreMesh(core_axis_name="c", subcore_axis_name="s")

    @functools.partial(
        pl.kernel,
        mesh=mesh,
        out_type=jax.ShapeDtypeStruct((NE, wrow), dt),
        scratch_types=[
            pltpu.VMEM_SHARED((NPAD, wrow), dt),
            pltpu.VMEM((ROWS_W,), jnp.int32),
        ] + [pltpu.VMEM((chunk, wrow), dt) for _ in range(nbuf)]
          + [pltpu.SemaphoreType.DMA for _ in range(2 * nbuf)],
    )
    def gather_k(table_hbm, idx_hbm, out_hbm, tab_s, idx_v, *bufs_and_sems):
        bufs = bufs_and_sems[:nbuf]
        sem_g = bufs_and_sems[nbuf:2 * nbuf]
        sem_o = bufs_and_sems[2 * nbuf:]
        sid = lax.axis_index("s")
        wid = sid * 2 + lax.axis_index("c")
        base = wid * ROWS_W
        pltpu.sync_copy(table_hbm.at[pl.ds(sid * stage, stage)],
                        tab_s.at[pl.ds(sid * stage, stage)])
        pltpu.sync_copy(idx_hbm.at[pl.ds(base, ROWS_W)], idx_v)
        plsc.subcore_barrier()

        g_h, o_h = {}, {}

        def start_g(c):
            g_h[c] = pltpu.async_copy(
                tab_s.at[idx_v.at[pl.ds(c * chunk, chunk)]],
                bufs[c % nbuf], sem_g[c % nbuf])

        def start_o(c):
            o_h[c] = pltpu.async_copy(
                bufs[c % nbuf], out_hbm.at[pl.ds(base + c * chunk, chunk)],
                sem_o[c % nbuf])

        for c in range(min(nbuf - 1, nch)):
            start_g(c)
        for c in range(nch):
            g_h[c].wait()
            start_o(c)
            nxt = c + nbuf - 1
            if nxt < nch:
                if nxt >= nbuf:
                    o_h[nxt - nbuf].wait()
                start_g(nxt)
        for c in range(max(0, nch - nbuf), nch):
            o_h[c].wait()

    return gather_k(table, idx)


def _sc_gather_two(tab_a, tab_b, idx, wrow, chunk, nbuf):
    dt = tab_a.dtype
    nch = ROWS_W // chunk
    stage = NPAD // 16
    mesh = plsc.VectorSubcoreMesh(core_axis_name="c", subcore_axis_name="s")

    @functools.partial(
        pl.kernel,
        mesh=mesh,
        out_type=[jax.ShapeDtypeStruct((NE, wrow), dt),
                  jax.ShapeDtypeStruct((NE, wrow), dt)],
        scratch_types=[
            pltpu.VMEM_SHARED((NPAD, wrow), dt),
            pltpu.VMEM((ROWS_W,), jnp.int32),
        ] + [pltpu.VMEM((chunk, wrow), dt) for _ in range(nbuf)]
          + [pltpu.SemaphoreType.DMA for _ in range(2 * nbuf)],
    )
    def gather_k(ta_hbm, tb_hbm, idx_hbm, oa_hbm, ob_hbm, tab_s, idx_v,
                 *bufs_and_sems):
        bufs = bufs_and_sems[:nbuf]
        sem_g = bufs_and_sems[nbuf:2 * nbuf]
        sem_o = bufs_and_sems[2 * nbuf:]
        sid = lax.axis_index("s")
        wid = sid * 2 + lax.axis_index("c")
        base = wid * ROWS_W
        pltpu.sync_copy(idx_hbm.at[pl.ds(base, ROWS_W)], idx_v)

        def phase(t_hbm, out_hbm):
            pltpu.sync_copy(t_hbm.at[pl.ds(sid * stage, stage)],
                            tab_s.at[pl.ds(sid * stage, stage)])
            plsc.subcore_barrier()
            g_h, o_h = {}, {}

            def start_g(c):
                g_h[c] = pltpu.async_copy(
                    tab_s.at[idx_v.at[pl.ds(c * chunk, chunk)]],
                    bufs[c % nbuf], sem_g[c % nbuf])

            def start_o(c):
                o_h[c] = pltpu.async_copy(
                    bufs[c % nbuf],
                    out_hbm.at[pl.ds(base + c * chunk, chunk)],
                    sem_o[c % nbuf])

            for c in range(min(nbuf - 1, nch)):
                start_g(c)
            for c in range(nch):
                g_h[c].wait()
                start_o(c)
                nxt = c + nbuf - 1
                if nxt < nch:
                    if nxt >= nbuf:
                        o_h[nxt - nbuf].wait()
                    start_g(nxt)
            for c in range(max(0, nch - nbuf), nch):
                o_h[c].wait()
            plsc.subcore_barrier()

        phase(ta_hbm, oa_hbm)
        phase(tb_hbm, ob_hbm)

    return gather_k(tab_a, tab_b, idx)


def _pre_body(inp_ref, im_ref, igd_ref, ogd_ref, lw_ref, lb_ref,
              x0_ref, ing_ref, outg_ref):
    x = jnp.dot(inp_ref[...], lw_ref[...], preferred_element_type=jnp.float32)
    x0_ref[...] = jax.nn.gelu(x + lb_ref[...])
    im = im_ref[...]
    ing_ref[...] = im + igd_ref[...]
    outg_ref[...] = im + ogd_ref[...]


def _compact_body(pw_ref, pj_ref):
    for k in range(K):
        pj_ref[:, k * 8:(k + 1) * 8] = pw_ref[:, k * D:k * D + 8]


def _layer_core(og_ref, pj_fn, xj_ref, x_ref, k1a_ref, k1b_ref, k2_ref,
                k3_ref, w_ref, b_ref):
    qp = jnp.dot(og_ref[...], k1a_ref[...], preferred_element_type=jnp.float32)
    k1b = k1b_ref[...]
    k2 = k2_ref[...]
    k3 = k3_ref[...]
    acc = jnp.zeros((BN, D), jnp.float32)
    for kk in range(K // 2):
        pj_p = pj_fn(kk)
        h1 = jax.nn.gelu(qp + jnp.dot(pj_p, k1b,
                                      preferred_element_type=jnp.float32))
        h2 = jax.nn.gelu(jnp.dot(h1, k2, preferred_element_type=jnp.float32))
        kap2 = jnp.dot(h2, k3, preferred_element_type=jnp.float32)
        acc = (acc
               + kap2[:, :D] * xj_ref[:, (2 * kk) * D:(2 * kk + 1) * D]
               + kap2[:, D:] * xj_ref[:, (2 * kk + 1) * D:(2 * kk + 2) * D])
    msg = acc * (1.0 / K)
    return jax.nn.gelu(jnp.dot(msg, w_ref[...],
                               preferred_element_type=jnp.float32)
                       + b_ref[...] + x_ref[...])


def _layer_body(og_ref, pj_ref, xj_ref, x_ref, k1a_ref, k1b_ref, k2_ref,
                k3_ref, w_ref, b_ref, out_ref):
    pj_fn = lambda kk: pj_ref[:, kk * 16:(kk + 1) * 16]
    out_ref[...] = _layer_core(og_ref, pj_fn, xj_ref, x_ref, k1a_ref,
                               k1b_ref, k2_ref, k3_ref, w_ref, b_ref)


def _layer_body_first(og_ref, pw_ref, xj_ref, x_ref, k1a_ref, k1b_ref,
                      k2_ref, k3_ref, w_ref, b_ref, out_ref, pj_ref):
    pj_fn = lambda kk: jnp.concatenate(
        [pw_ref[:, (2 * kk) * D:(2 * kk) * D + 8],
         pw_ref[:, (2 * kk + 1) * D:(2 * kk + 1) * D + 8]], axis=1)
    out_ref[...] = _layer_core(og_ref, pj_fn, xj_ref, x_ref, k1a_ref,
                               k1b_ref, k2_ref, k3_ref, w_ref, b_ref)
    for k in range(K):
        pj_ref[:, k * 8:(k + 1) * 8] = pw_ref[:, k * D:k * D + 8]


def _layer_body_last(og_ref, pj_ref, xj_ref, x_ref, k1a_ref, k1b_ref, k2_ref,
                     k3_ref, w_ref, b_ref, pw_ref, pb_ref, out_ref):
    pj_fn = lambda kk: pj_ref[:, kk * 16:(kk + 1) * 16]
    xn = _layer_core(og_ref, pj_fn, xj_ref, x_ref, k1a_ref, k1b_ref,
                     k2_ref, k3_ref, w_ref, b_ref)
    out_ref[...] = jnp.dot(xn, pw_ref[...],
                           preferred_element_type=jnp.float32) + pb_ref[...]


def _full(shape):
    return pl.BlockSpec(shape, lambda b: (0,) * len(shape))


def _rows(width):
    return pl.BlockSpec((BN, width), lambda b: (b, 0))


def _tc_pre(inp_p, im128, igd128, ogd128, lift_W, lift_b):
    return pl.pallas_call(
        _pre_body,
        grid=(GRID,),
        in_specs=[_rows(D), _rows(D), _rows(D), _rows(D),
                  _full((D, D)), _full((1, D))],
        out_specs=[_rows(D), _rows(D), _rows(D)],
        out_shape=[jax.ShapeDtypeStruct((NPAD, D), jnp.float32),
                   jax.ShapeDtypeStruct((NPAD, D), jnp.float32),
                   jax.ShapeDtypeStruct((NPAD, D), jnp.float32)],
    )(inp_p, im128, igd128, ogd128, lift_W, lift_b)


def _tc_compact(posw):
    return pl.pallas_call(
        _compact_body,
        grid=(GRID,),
        in_specs=[_rows(K * D)],
        out_specs=_rows(K * 8),
        out_shape=jax.ShapeDtypeStruct((NPAD, K * 8), jnp.float32),
    )(posw)


def _tc_layer_first(og, posw, xjv, x, k1a, k1b, k2, k3, w, b):
    return pl.pallas_call(
        _layer_body_first,
        grid=(GRID,),
        in_specs=[_rows(D), _rows(K * D), _rows(K * D), _rows(D),
                  _full((D, D)), _full((16, D)), _full((D, D)), _full((D, 2 * D)),
                  _full((D, D)), _full((1, D))],
        out_specs=[_rows(D), _rows(K * 8)],
        out_shape=[jax.ShapeDtypeStruct((NPAD, D), jnp.float32),
                   jax.ShapeDtypeStruct((NPAD, K * 8), jnp.float32)],
    )(og, posw, xjv, x, k1a, k1b, k2, k3, w, b)


def _tc_layer(og, pjv, xjv, x, k1a, k1b, k2, k3, w, b):
    return pl.pallas_call(
        _layer_body,
        grid=(GRID,),
        in_specs=[_rows(D), _rows(K * 8), _rows(K * D), _rows(D),
                  _full((D, D)), _full((16, D)), _full((D, D)), _full((D, 2 * D)),
                  _full((D, D)), _full((1, D))],
        out_specs=_rows(D),
        out_shape=jax.ShapeDtypeStruct((NPAD, D), jnp.float32),
    )(og, pjv, xjv, x, k1a, k1b, k2, k3, w, b)


def _tc_layer_last(og, pjv, xjv, x, k1a, k1b, k2, k3, w, b, pw, pb):
    return pl.pallas_call(
        _layer_body_last,
        grid=(GRID,),
        in_specs=[_rows(D), _rows(K * 8), _rows(K * D), _rows(D),
                  _full((D, D)), _full((16, D)), _full((D, D)), _full((D, 2 * D)),
                  _full((D, D)), _full((1, D)), _full((D, D)), _full((1, D))],
        out_specs=_rows(D),
        out_shape=jax.ShapeDtypeStruct((NPAD, D), jnp.float32),
    )(og, pjv, xjv, x, k1a, k1b, k2, k3, w, b, pw, pb)


def kernel(inp, out_grid_displacement, in_grid_displacement, neighbor_idx,
           initial_mesh, lift_W, lift_b, K1, K2, K3, Wl, bl, proj_W, proj_b):
    f32 = jnp.float32

    def pad_n(a, width):
        out = jnp.zeros((NPAD, width), f32)
        return out.at[:N, :a.shape[1]].set(a)

    inp_p = pad_n(inp[0], D)
    im128 = pad_n(initial_mesh, D)
    igd128 = pad_n(in_grid_displacement, D)
    ogd128 = pad_n(out_grid_displacement, D)

    idx_flat = jnp.zeros((NE,), jnp.int32)
    idx_flat = idx_flat.at[:N * K].set(neighbor_idx.reshape(-1))

    lb = lift_b.reshape(1, D)
    pb = proj_b.reshape(1, D)

    k1a_ = jnp.zeros((L, D, H), f32).at[:, :ND, :].set(K1[:, :ND, :])
    k1a = jnp.concatenate([k1a_, k1a_], axis=2)
    k1b_ = jnp.zeros((L, 8, H), f32).at[:, :ND, :].set(K1[:, ND:, :])
    k1b = jnp.zeros((L, 16, D), f32)
    k1b = k1b.at[:, :8, :H].set(k1b_).at[:, 8:, H:].set(k1b_)
    k2p = jnp.zeros((L, D, D), f32)
    k2p = k2p.at[:, :H, :H].set(K2).at[:, H:, H:].set(K2)
    k3p = jnp.zeros((L, D, 2 * D), f32)
    k3p = k3p.at[:, :H, :D].set(K3).at[:, H:, D:].set(K3)

    x, ing, outg = _tc_pre(inp_p, im128, igd128, ogd128, lift_W, lb)

    posw, xj0 = _sc_gather_two(ing, x, idx_flat, D, 160, 2)
    x, pjv = _tc_layer_first(ing, posw.reshape(NPAD, K * D),
                             xj0.reshape(NPAD, K * D), x, k1a[0], k1b[0],
                             k2p[0], k3p[0], Wl[0], bl[0].reshape(1, D))

    for i in range(1, L):
        xj = _sc_gather_spmem(x, idx_flat, D, 160, 2)
        xjv = xj.reshape(NPAD, K * D)
        og = outg if i == L - 1 else ing
        if i == L - 1:
            x = _tc_layer_last(og, pjv, xjv, x, k1a[i], k1b[i], k2p[i], k3p[i],
                               Wl[i], bl[i].reshape(1, D), proj_W, pb)
        else:
            x = _tc_layer(og, pjv, xjv, x, k1a[i], k1b[i], k2p[i], k3p[i],
                          Wl[i], bl[i].reshape(1, D))

    return x[:N][None]


# speedup vs baseline: 2.5511x; 1.0090x over previous
import functools

import jax
import jax.numpy as jnp
from jax import lax
from jax.experimental import pallas as pl
from jax.experimental.pallas import tpu as pltpu
from jax.experimental.pallas import tpu_sc as plsc

N = 10000
K = 16
D = 128
ND = 3
L = 4
H = 64

NPAD = 10240
NE = NPAD * K

BN = 512
GRID = NPAD // BN

NW = 32
ROWS_W = NE // NW


def _sc_gather(table, idx, wrow, chunk, nbuf):
    dt = table.dtype
    nch = ROWS_W // chunk
    mesh = plsc.VectorSubcoreMesh(core_axis_name="c", subcore_axis_name="s")

    @functools.partial(
        pl.kernel,
        mesh=mesh,
        out_type=jax.ShapeDtypeStruct((NE, wrow), dt),
        scratch_types=[
            pltpu.VMEM((ROWS_W,), jnp.int32),
        ] + [pltpu.VMEM((chunk, wrow), dt) for _ in range(nbuf)]
          + [pltpu.SemaphoreType.DMA for _ in range(2 * nbuf)],
    )
    def gather_k(table_hbm, idx_hbm, out_hbm, idx_v, *bufs_and_sems):
        bufs = bufs_and_sems[:nbuf]
        sem_g = bufs_and_sems[nbuf:2 * nbuf]
        sem_o = bufs_and_sems[2 * nbuf:]
        wid = lax.axis_index("s") * 2 + lax.axis_index("c")
        base = wid * ROWS_W
        pltpu.sync_copy(idx_hbm.at[pl.ds(base, ROWS_W)], idx_v)

        g_h, o_h = {}, {}

        def start_g(c):
            g_h[c] = pltpu.async_copy(
                table_hbm.at[idx_v.at[pl.ds(c * chunk, chunk)]],
                bufs[c % nbuf], sem_g[c % nbuf])

        def start_o(c):
            o_h[c] = pltpu.async_copy(
                bufs[c % nbuf], out_hbm.at[pl.ds(base + c * chunk, chunk)],
                sem_o[c % nbuf])

        for c in range(min(nbuf - 1, nch)):
            start_g(c)
        for c in range(nch):
            g_h[c].wait()
            start_o(c)
            nxt = c + nbuf - 1
            if nxt < nch:
                if nxt >= nbuf:
                    o_h[nxt - nbuf].wait()
                start_g(nxt)
        for c in range(max(0, nch - nbuf), nch):
            o_h[c].wait()

    return gather_k(table, idx)


def _sc_gather_spmem(table, idx, wrow, chunk, nbuf):
    dt = table.dtype
    nch = ROWS_W // chunk
    stage = NPAD // 16
    mesh = plsc.VectorSubcoreMesh(core_axis_name="c", subcore_axis_name="s")

    @functools.partial(
        pl.kernel,
        mesh=mesh,
        out_type=jax.ShapeDtypeStruct((NE, wrow), dt),
        scratch_types=[
            pltpu.VMEM_SHARED((NPAD, wrow), dt),
            pltpu.VMEM((ROWS_W,), jnp.int32),
        ] + [pltpu.VMEM((chunk, wrow), dt) for _ in range(nbuf)]
          + [pltpu.SemaphoreType.DMA for _ in range(2 * nbuf)],
    )
    def gather_k(table_hbm, idx_hbm, out_hbm, tab_s, idx_v, *bufs_and_sems):
        bufs = bufs_and_sems[:nbuf]
        sem_g = bufs_and_sems[nbuf:2 * nbuf]
        sem_o = bufs_and_sems[2 * nbuf:]
        sid = lax.axis_index("s")
        wid = sid * 2 + lax.axis_index("c")
        base = wid * ROWS_W
        pltpu.sync_copy(table_hbm.at[pl.ds(sid * stage, stage)],
                        tab_s.at[pl.ds(sid * stage, stage)])
        pltpu.sync_copy(idx_hbm.at[pl.ds(base, ROWS_W)], idx_v)
        plsc.subcore_barrier()

        g_h, o_h = {}, {}

        def start_g(c):
            g_h[c] = pltpu.async_copy(
                tab_s.at[idx_v.at[pl.ds(c * chunk, chunk)]],
                bufs[c % nbuf], sem_g[c % nbuf])

        def start_o(c):
            o_h[c] = pltpu.async_copy(
                bufs[c % nbuf], out_hbm.at[pl.ds(base + c * chunk, chunk)],
                sem_o[c % nbuf])

        for c in range(min(nbuf - 1, nch)):
            start_g(c)
        for c in range(nch):
            g_h[c].wait()
            start_o(c)
            nxt = c + nbuf - 1
            if nxt < nch:
                if nxt >= nbuf:
                    o_h[nxt - nbuf].wait()
                start_g(nxt)
        for c in range(max(0, nch - nbuf), nch):
            o_h[c].wait()

    return gather_k(table, idx)


def _sc_gather_two(tab_a, tab_b, idx, wrow, chunk, nbuf):
    dt = tab_a.dtype
    nch = ROWS_W // chunk
    stage = NPAD // 16
    mesh = plsc.VectorSubcoreMesh(core_axis_name="c", subcore_axis_name="s")

    @functools.partial(
        pl.kernel,
        mesh=mesh,
        out_type=[jax.ShapeDtypeStruct((NE, wrow), dt),
                  jax.ShapeDtypeStruct((NE, wrow), dt)],
        scratch_types=[
            pltpu.VMEM_SHARED((NPAD, wrow), dt),
            pltpu.VMEM((ROWS_W,), jnp.int32),
        ] + [pltpu.VMEM((chunk, wrow), dt) for _ in range(nbuf)]
          + [pltpu.SemaphoreType.DMA for _ in range(2 * nbuf)],
    )
    def gather_k(ta_hbm, tb_hbm, idx_hbm, oa_hbm, ob_hbm, tab_s, idx_v,
                 *bufs_and_sems):
        bufs = bufs_and_sems[:nbuf]
        sem_g = bufs_and_sems[nbuf:2 * nbuf]
        sem_o = bufs_and_sems[2 * nbuf:]
        sid = lax.axis_index("s")
        wid = sid * 2 + lax.axis_index("c")
        base = wid * ROWS_W
        pltpu.sync_copy(idx_hbm.at[pl.ds(base, ROWS_W)], idx_v)

        def phase(t_hbm, out_hbm):
            pltpu.sync_copy(t_hbm.at[pl.ds(sid * stage, stage)],
                            tab_s.at[pl.ds(sid * stage, stage)])
            plsc.subcore_barrier()
            g_h, o_h = {}, {}

            def start_g(c):
                g_h[c] = pltpu.async_copy(
                    tab_s.at[idx_v.at[pl.ds(c * chunk, chunk)]],
                    bufs[c % nbuf], sem_g[c % nbuf])

            def start_o(c):
                o_h[c] = pltpu.async_copy(
                    bufs[c % nbuf],
                    out_hbm.at[pl.ds(base + c * chunk, chunk)],
                    sem_o[c % nbuf])

            for c in range(min(nbuf - 1, nch)):
                start_g(c)
            for c in range(nch):
                g_h[c].wait()
                start_o(c)
                nxt = c + nbuf - 1
                if nxt < nch:
                    if nxt >= nbuf:
                        o_h[nxt - nbuf].wait()
                    start_g(nxt)
            for c in range(max(0, nch - nbuf), nch):
                o_h[c].wait()
            plsc.subcore_barrier()

        phase(ta_hbm, oa_hbm)
        phase(tb_hbm, ob_hbm)

    return gather_k(tab_a, tab_b, idx)


def _pre_body(inp_ref, im_ref, igd_ref, ogd_ref, lw_ref, lb_ref,
              x0_ref, ing_ref, outg_ref):
    x = jnp.dot(inp_ref[...], lw_ref[...], preferred_element_type=jnp.float32)
    x0_ref[...] = jax.nn.gelu(x + lb_ref[...])
    im = im_ref[...]
    ing_ref[...] = im + igd_ref[...]
    outg_ref[...] = im + ogd_ref[...]


def _compact_body(pw_ref, pj_ref):
    for k in range(K):
        pj_ref[:, k * 8:(k + 1) * 8] = pw_ref[:, k * D:k * D + 8]


def _layer_core(og_ref, pj_fn, xj_ref, x_ref, k1a_ref, k1b_ref, k2_ref,
                k3_ref, w_ref, b_ref):
    qp = jnp.dot(og_ref[...], k1a_ref[...], preferred_element_type=jnp.float32)
    k1b = k1b_ref[...]
    k2 = k2_ref[...]
    k3 = k3_ref[...]
    acc = jnp.zeros((BN, D), jnp.float32)
    for kk in range(K // 2):
        pj_p = pj_fn(kk)
        h1 = jax.nn.gelu(qp + jnp.dot(pj_p, k1b,
                                      preferred_element_type=jnp.float32))
        h2 = jax.nn.gelu(jnp.dot(h1, k2, preferred_element_type=jnp.float32))
        kap2 = jnp.dot(h2, k3, preferred_element_type=jnp.float32)
        acc = (acc
               + kap2[:, :D] * xj_ref[:, (2 * kk) * D:(2 * kk + 1) * D]
               + kap2[:, D:] * xj_ref[:, (2 * kk + 1) * D:(2 * kk + 2) * D])
    msg = acc * (1.0 / K)
    return jax.nn.gelu(jnp.dot(msg, w_ref[...],
                               preferred_element_type=jnp.float32)
                       + b_ref[...] + x_ref[...])


def _layer_body(og_ref, pj_ref, xj_ref, x_ref, k1a_ref, k1b_ref, k2_ref,
                k3_ref, w_ref, b_ref, out_ref):
    pj_fn = lambda kk: pj_ref[:, kk * 16:(kk + 1) * 16]
    out_ref[...] = _layer_core(og_ref, pj_fn, xj_ref, x_ref, k1a_ref,
                               k1b_ref, k2_ref, k3_ref, w_ref, b_ref)


def _layer_body_first(og_ref, pw_ref, xj_ref, x_ref, k1a_ref, k1b_ref,
                      k2_ref, k3_ref, w_ref, b_ref, out_ref, pj_ref):
    pj_fn = lambda kk: jnp.concatenate(
        [pw_ref[:, (2 * kk) * D:(2 * kk) * D + 8],
         pw_ref[:, (2 * kk + 1) * D:(2 * kk + 1) * D + 8]], axis=1)
    out_ref[...] = _layer_core(og_ref, pj_fn, xj_ref, x_ref, k1a_ref,
                               k1b_ref, k2_ref, k3_ref, w_ref, b_ref)
    for k in range(K):
        pj_ref[:, k * 8:(k + 1) * 8] = pw_ref[:, k * D:k * D + 8]


def _layer_body_last(og_ref, pj_ref, xj_ref, x_ref, k1a_ref, k1b_ref, k2_ref,
                     k3_ref, w_ref, b_ref, pw_ref, pb_ref, out_ref):
    pj_fn = lambda kk: pj_ref[:, kk * 16:(kk + 1) * 16]
    xn = _layer_core(og_ref, pj_fn, xj_ref, x_ref, k1a_ref, k1b_ref,
                     k2_ref, k3_ref, w_ref, b_ref)
    out_ref[...] = jnp.dot(xn, pw_ref[...],
                           preferred_element_type=jnp.float32) + pb_ref[...]


def _full(shape):
    return pl.BlockSpec(shape, lambda b: (0,) * len(shape))


def _rows(width):
    return pl.BlockSpec((BN, width), lambda b: (b, 0))


def _tc_pre(inp_p, im128, igd128, ogd128, lift_W, lift_b):
    return pl.pallas_call(
        _pre_body,
        grid=(GRID,),
        in_specs=[_rows(D), _rows(D), _rows(D), _rows(D),
                  _full((D, D)), _full((1, D))],
        out_specs=[_rows(D), _rows(D), _rows(D)],
        out_shape=[jax.ShapeDtypeStruct((NPAD, D), jnp.float32),
                   jax.ShapeDtypeStruct((NPAD, D), jnp.float32),
                   jax.ShapeDtypeStruct((NPAD, D), jnp.float32)],
    )(inp_p, im128, igd128, ogd128, lift_W, lift_b)


def _tc_compact(posw):
    return pl.pallas_call(
        _compact_body,
        grid=(GRID,),
        in_specs=[_rows(K * D)],
        out_specs=_rows(K * 8),
        out_shape=jax.ShapeDtypeStruct((NPAD, K * 8), jnp.float32),
    )(posw)


def _tc_layer_first(og, posw, xjv, x, k1a, k1b, k2, k3, w, b):
    return pl.pallas_call(
        _layer_body_first,
        grid=(GRID,),
        in_specs=[_rows(D), _rows(K * D), _rows(K * D), _rows(D),
                  _full((D, D)), _full((16, D)), _full((D, D)), _full((D, 2 * D)),
                  _full((D, D)), _full((1, D))],
        out_specs=[_rows(D), _rows(K * 8)],
        out_shape=[jax.ShapeDtypeStruct((NPAD, D), jnp.float32),
                   jax.ShapeDtypeStruct((NPAD, K * 8), jnp.float32)],
    )(og, posw, xjv, x, k1a, k1b, k2, k3, w, b)


def _tc_layer(og, pjv, xjv, x, k1a, k1b, k2, k3, w, b):
    return pl.pallas_call(
        _layer_body,
        grid=(GRID,),
        in_specs=[_rows(D), _rows(K * 8), _rows(K * D), _rows(D),
                  _full((D, D)), _full((16, D)), _full((D, D)), _full((D, 2 * D)),
                  _full((D, D)), _full((1, D))],
        out_specs=_rows(D),
        out_shape=jax.ShapeDtypeStruct((NPAD, D), jnp.float32),
    )(og, pjv, xjv, x, k1a, k1b, k2, k3, w, b)


def _tc_layer_last(og, pjv, xjv, x, k1a, k1b, k2, k3, w, b, pw, pb):
    return pl.pallas_call(
        _layer_body_last,
        grid=(GRID,),
        in_specs=[_rows(D), _rows(K * 8), _rows(K * D), _rows(D),
                  _full((D, D)), _full((16, D)), _full((D, D)), _full((D, 2 * D)),
                  _full((D, D)), _full((1, D)), _full((D, D)), _full((1, D))],
        out_specs=_rows(D),
        out_shape=jax.ShapeDtypeStruct((NPAD, D), jnp.float32),
    )(og, pjv, xjv, x, k1a, k1b, k2, k3, w, b, pw, pb)


def kernel(inp, out_grid_displacement, in_grid_displacement, neighbor_idx,
           initial_mesh, lift_W, lift_b, K1, K2, K3, Wl, bl, proj_W, proj_b):
    f32 = jnp.float32

    def pad_n(a, width):
        out = jnp.zeros((NPAD, width), f32)
        return out.at[:N, :a.shape[1]].set(a)

    inp_p = pad_n(inp[0], D)
    im128 = pad_n(initial_mesh, D)
    igd128 = pad_n(in_grid_displacement, D)
    ogd128 = pad_n(out_grid_displacement, D)

    idx_flat = jnp.zeros((NE,), jnp.int32)
    idx_flat = idx_flat.at[:N * K].set(neighbor_idx.reshape(-1))

    lb = lift_b.reshape(1, D)
    pb = proj_b.reshape(1, D)

    k1a_ = jnp.zeros((L, D, H), f32).at[:, :ND, :].set(K1[:, :ND, :])
    k1a = jnp.concatenate([k1a_, k1a_], axis=2)
    k1b_ = jnp.zeros((L, 8, H), f32).at[:, :ND, :].set(K1[:, ND:, :])
    k1b = jnp.zeros((L, 16, D), f32)
    k1b = k1b.at[:, :8, :H].set(k1b_).at[:, 8:, H:].set(k1b_)
    k2p = jnp.zeros((L, D, D), f32)
    k2p = k2p.at[:, :H, :H].set(K2).at[:, H:, H:].set(K2)
    k3p = jnp.zeros((L, D, 2 * D), f32)
    k3p = k3p.at[:, :H, :D].set(K3).at[:, H:, D:].set(K3)

    x, ing, outg = _tc_pre(inp_p, im128, igd128, ogd128, lift_W, lb)

    posw, xj0 = _sc_gather_two(ing, x, idx_flat, D, 80, 4)
    x, pjv = _tc_layer_first(ing, posw.reshape(NPAD, K * D),
                             xj0.reshape(NPAD, K * D), x, k1a[0], k1b[0],
                             k2p[0], k3p[0], Wl[0], bl[0].reshape(1, D))

    for i in range(1, L):
        xj = _sc_gather_spmem(x, idx_flat, D, 80, 4)
        xjv = xj.reshape(NPAD, K * D)
        og = outg if i == L - 1 else ing
        if i == L - 1:
            x = _tc_layer_last(og, pjv, xjv, x, k1a[i], k1b[i], k2p[i], k3p[i],
                               Wl[i], bl[i].reshape(1, D), proj_W, pb)
        else:
            x = _tc_layer(og, pjv, xjv, x, k1a[i], k1b[i], k2p[i], k3p[i],
                          Wl[i], bl[i].reshape(1, D))

    return x[:N][None]
